# baseline (speedup 1.0000x reference)
import functools

import jax
import jax.numpy as jnp
from jax import lax
from jax.experimental import pallas as pl
from jax.experimental.pallas import tpu as pltpu
from jax.experimental.pallas import tpu_sc as plsc

N = 10000
E = 320000
D_IN = 128
HID = 64
NC = 2
NS = 16
NW = NC * NS
EPW = E // NW
C = 125
NCHUNK = EPW // C
NBUF = 5
RPT = 624
RREM = N - NS * RPT

_MESH = plsc.VectorSubcoreMesh(core_axis_name="c", subcore_axis_name="s")


def _sc_degree(dstr, zeros_n):
    @functools.partial(
        pl.kernel,
        out_type=jax.ShapeDtypeStruct((NC, N), jnp.float32),
        mesh=_MESH,
        compiler_params=pltpu.CompilerParams(use_tc_tiling_on_sc=False),
        scratch_types=[
            pltpu.VMEM((NCHUNK, C), jnp.int32),
            pltpu.VMEM((128,), jnp.float32),
            pltpu.VMEM_SHARED((N,), jnp.float32),
        ],
    )
    def k(dstr_hbm, zeros_hbm, out_hbm, didx_all, ones_v, deg_sh):
        c = lax.axis_index("c")
        s = lax.axis_index("s")
        wid = s * NC + c
        @pl.when(s == 0)
        def _():
            pltpu.sync_copy(zeros_hbm, deg_sh)

        pltpu.sync_copy(dstr_hbm.at[wid], didx_all)
        for i in range(8):
            ones_v[pl.ds(i * 16, 16)] = jnp.full((16,), 1.0, jnp.float32)
        plsc.subcore_barrier()

        def body(j, carry):
            pltpu.sync_copy(ones_v.at[pl.ds(0, C)],
                            deg_sh.at[didx_all.at[j]], add=True)
            return carry

        lax.fori_loop(0, NCHUNK, body, 0)
        plsc.subcore_barrier()

        @pl.when(s == 0)
        def _():
            pltpu.sync_copy(deg_sh, out_hbm.at[c])

    return k(dstr, zeros_n)


def _sc_spmm(hs, srcr, dstr, zeros_nh):
    @functools.partial(
        pl.kernel,
        out_type=jax.ShapeDtypeStruct((NC, N, HID), jnp.float32),
        mesh=_MESH,
        compiler_params=pltpu.CompilerParams(use_tc_tiling_on_sc=False),
        scratch_types=[
            pltpu.VMEM((NCHUNK, C), jnp.int32),
            pltpu.VMEM((NCHUNK, C), jnp.int32),
        ]
        + [pltpu.VMEM((C, HID), jnp.float32) for _ in range(NBUF)]
        + [pltpu.VMEM_SHARED((N, HID), jnp.float32)]
        + [pltpu.SemaphoreType.DMA for _ in range(2 * NBUF)],
    )
    def k(hs_hbm, srcr_hbm, dstr_hbm, zeros_hbm, out_hbm,
          sidx_all, didx_all, *rest):
        rows = rest[:NBUF]
        acc_sh = rest[NBUF]
        gsems = rest[NBUF + 1:NBUF + 1 + NBUF]
        ssems = rest[NBUF + 1 + NBUF:]
        c = lax.axis_index("c")
        s = lax.axis_index("s")
        wid = s * NC + c
        r0 = s * RPT
        pltpu.sync_copy(zeros_hbm.at[pl.ds(r0, RPT)], acc_sh.at[pl.ds(r0, RPT)])

        @pl.when(s == 0)
        def _():
            pltpu.sync_copy(zeros_hbm.at[pl.ds(NS * RPT, RREM)],
                            acc_sh.at[pl.ds(NS * RPT, RREM)])

        pltpu.sync_copy(srcr_hbm.at[wid], sidx_all)
        pltpu.sync_copy(dstr_hbm.at[wid], didx_all)
        plsc.subcore_barrier()

        for b in range(NBUF):
            pltpu.async_copy(hs_hbm.at[sidx_all.at[b]], rows[b], gsems[b])

        def body(g, carry):
            j0 = g * NBUF
            for b in range(NBUF):
                pltpu.make_async_copy(
                    hs_hbm.at[pl.ds(0, C)], rows[b], gsems[b]).wait()
                pltpu.async_copy(
                    rows[b], acc_sh.at[didx_all.at[j0 + b]], ssems[b],
                    add=True)
            for b in range(NBUF):
                pltpu.make_async_copy(
                    hs_hbm.at[pl.ds(0, C)], rows[b], ssems[b]).wait()
                pltpu.async_copy(
                    hs_hbm.at[sidx_all.at[j0 + NBUF + b]], rows[b], gsems[b])
            return carry

        lax.fori_loop(0, NCHUNK // NBUF - 1, body, 0)

        jt = NCHUNK - NBUF
        for b in range(NBUF):
            pltpu.make_async_copy(hs_hbm.at[pl.ds(0, C)], rows[b], gsems[b]).wait()
            pltpu.async_copy(
                rows[b], acc_sh.at[didx_all.at[jt + b]], ssems[b], add=True)
        for b in range(NBUF):
            pltpu.make_async_copy(
                hs_hbm.at[pl.ds(0, C)], rows[b], ssems[b]).wait()

        plsc.subcore_barrier()
        pltpu.sync_copy(acc_sh.at[pl.ds(r0, RPT)], out_hbm.at[c, pl.ds(r0, RPT)])

        @pl.when(s == 0)
        def _():
            pltpu.sync_copy(acc_sh.at[pl.ds(NS * RPT, RREM)],
                            out_hbm.at[c, pl.ds(NS * RPT, RREM)])

    return k(hs, srcr, dstr, zeros_nh)


BLK = 1000


def _tc_first(deg_t, x, w1):
    def body(deg_ref, x_ref, w_ref, hs_ref, dinv_ref):
        deg = deg_ref[:, 0:1] + deg_ref[:, 1:2] + 1.0
        dinv = lax.rsqrt(deg)
        hw = jnp.dot(x_ref[...], w_ref[...], preferred_element_type=jnp.float32)
        hs_ref[...] = hw * dinv
        dinv_ref[...] = dinv

    return pl.pallas_call(
        body,
        grid=(N // BLK,),
        in_specs=[
            pl.BlockSpec((BLK, NC), lambda i: (i, 0)),
            pl.BlockSpec((BLK, D_IN), lambda i: (i, 0)),
            pl.BlockSpec((D_IN, HID), lambda i: (0, 0)),
        ],
        out_specs=[
            pl.BlockSpec((BLK, HID), lambda i: (i, 0)),
            pl.BlockSpec((BLK, 1), lambda i: (i, 0)),
        ],
        out_shape=[
            jax.ShapeDtypeStruct((N, HID), jnp.float32),
            jax.ShapeDtypeStruct((N, 1), jnp.float32),
        ],
    )(deg_t, x, w1)


def _tc_mid(acc, hs, dinv, b_row, w_next):
    def body(acc_ref, hs_ref, dinv_ref, b_ref, w_ref, out_ref):
        t = acc_ref[0] + acc_ref[1] + hs_ref[...]
        h = jnp.maximum(t * dinv_ref[...] + b_ref[...], 0.0)
        hw = jnp.dot(h, w_ref[...], preferred_element_type=jnp.float32)
        out_ref[...] = hw * dinv_ref[...]

    return pl.pallas_call(
        body,
        grid=(N // BLK,),
        in_specs=[
            pl.BlockSpec((NC, BLK, HID), lambda i: (0, i, 0)),
            pl.BlockSpec((BLK, HID), lambda i: (i, 0)),
            pl.BlockSpec((BLK, 1), lambda i: (i, 0)),
            pl.BlockSpec((1, HID), lambda i: (0, 0)),
            pl.BlockSpec((HID, HID), lambda i: (0, 0)),
        ],
        out_specs=pl.BlockSpec((BLK, HID), lambda i: (i, 0)),
        out_shape=jax.ShapeDtypeStruct((N, HID), jnp.float32),
    )(acc, hs, dinv, b_row, w_next)


def _tc_last(acc, hs, dinv, b_row, fc_w, fc_b_row):
    def body(acc_ref, hs_ref, dinv_ref, b_ref, fcw_ref, fcb_ref, out_ref):
        t = acc_ref[0] + acc_ref[1] + hs_ref[...]
        h = jnp.maximum(t * dinv_ref[...] + b_ref[...], 0.0)
        out_ref[...] = (
            jnp.dot(h, fcw_ref[...], preferred_element_type=jnp.float32)
            + fcb_ref[...]
        )

    return pl.pallas_call(
        body,
        grid=(N // BLK,),
        in_specs=[
            pl.BlockSpec((NC, BLK, HID), lambda i: (0, i, 0)),
            pl.BlockSpec((BLK, HID), lambda i: (i, 0)),
            pl.BlockSpec((BLK, 1), lambda i: (i, 0)),
            pl.BlockSpec((1, HID), lambda i: (0, 0)),
            pl.BlockSpec((HID, 1), lambda i: (0, 0)),
            pl.BlockSpec((1, 1), lambda i: (0, 0)),
        ],
        out_specs=pl.BlockSpec((BLK, 1), lambda i: (i, 0)),
        out_shape=jax.ShapeDtypeStruct((N, 1), jnp.float32),
    )(acc, hs, dinv, b_row, fc_w, fc_b_row)


def kernel(x, edge_index, W1, b1, W2, b2, W3, b3, fc_w, fc_b):
    srcr = edge_index[0].reshape(NW, NCHUNK, C)
    dstr = edge_index[1].reshape(NW, NCHUNK, C)
    zeros_n = jnp.zeros((N,), jnp.float32)
    zeros_nh = jnp.zeros((N, HID), jnp.float32)

    deg2 = _sc_degree(dstr, zeros_n)
    deg_t = deg2.T
    hs1, dinv = _tc_first(deg_t, x, W1)

    acc1 = _sc_spmm(hs1, srcr, dstr, zeros_nh)
    hs2 = _tc_mid(acc1, hs1, dinv, b1.reshape(1, HID), W2)
    acc2 = _sc_spmm(hs2, srcr, dstr, zeros_nh)
    hs3 = _tc_mid(acc2, hs2, dinv, b2.reshape(1, HID), W3)
    acc3 = _sc_spmm(hs3, srcr, dstr, zeros_nh)
    return _tc_last(acc3, hs3, dinv, b3.reshape(1, HID),
                    fc_w, fc_b.reshape(1, 1))


# speedup vs baseline: 1.0842x; 1.0842x over previous
import functools

import jax
import jax.numpy as jnp
from jax import lax
from jax.experimental import pallas as pl
from jax.experimental.pallas import tpu as pltpu
from jax.experimental.pallas import tpu_sc as plsc

N = 10000
E = 320000
D_IN = 128
HID = 64
NC = 2
NS = 16
NW = NC * NS
EPW = E // NW
C = 125
NCHUNK = EPW // C
NBUF = 5
RPT = 624
RREM = N - NS * RPT

_MESH = plsc.VectorSubcoreMesh(core_axis_name="c", subcore_axis_name="s")


def _sc_degree(dstr, zeros_n):
    @functools.partial(
        pl.kernel,
        out_type=jax.ShapeDtypeStruct((NC, N), jnp.float32),
        mesh=_MESH,
        compiler_params=pltpu.CompilerParams(use_tc_tiling_on_sc=False),
        scratch_types=[
            pltpu.VMEM((NCHUNK, C), jnp.int32),
            pltpu.VMEM((128,), jnp.float32),
            pltpu.VMEM_SHARED((N,), jnp.float32),
        ],
    )
    def k(dstr_hbm, zeros_hbm, out_hbm, didx_all, ones_v, deg_sh):
        c = lax.axis_index("c")
        s = lax.axis_index("s")
        wid = s * NC + c
        @pl.when(s == 0)
        def _():
            pltpu.sync_copy(zeros_hbm, deg_sh)

        pltpu.sync_copy(dstr_hbm.at[wid], didx_all)
        for i in range(8):
            ones_v[pl.ds(i * 16, 16)] = jnp.full((16,), 1.0, jnp.float32)
        plsc.subcore_barrier()

        def body(j, carry):
            pltpu.sync_copy(ones_v.at[pl.ds(0, C)],
                            deg_sh.at[didx_all.at[j]], add=True)
            return carry

        lax.fori_loop(0, NCHUNK, body, 0)
        plsc.subcore_barrier()

        @pl.when(s == 0)
        def _():
            pltpu.sync_copy(deg_sh, out_hbm.at[c])

    return k(dstr, zeros_n)


def _sc_spmm(hs, srcr, dstr):
    @functools.partial(
        pl.kernel,
        out_type=jax.ShapeDtypeStruct((NC, N, HID), jnp.float32),
        mesh=_MESH,
        compiler_params=pltpu.CompilerParams(use_tc_tiling_on_sc=False),
        scratch_types=[
            pltpu.VMEM((NCHUNK, C), jnp.int32),
            pltpu.VMEM((NCHUNK, C), jnp.int32),
        ]
        + [pltpu.VMEM((C, HID), jnp.float32) for _ in range(NBUF)]
        + [pltpu.VMEM_SHARED((N, HID), jnp.float32)]
        + [pltpu.SemaphoreType.DMA for _ in range(NBUF)],
    )
    def k(hs_hbm, srcr_hbm, dstr_hbm, out_hbm,
          sidx_all, didx_all, *rest):
        rows = rest[:NBUF]
        acc_sh = rest[NBUF]
        sems = rest[NBUF + 1:]
        c = lax.axis_index("c")
        s = lax.axis_index("s")
        wid = s * NC + c
        r0 = s * RPT
        pltpu.sync_copy(hs_hbm.at[pl.ds(r0, RPT)], acc_sh.at[pl.ds(r0, RPT)])

        @pl.when(s == 0)
        def _():
            pltpu.sync_copy(hs_hbm.at[pl.ds(NS * RPT, RREM)],
                            acc_sh.at[pl.ds(NS * RPT, RREM)])

        pltpu.sync_copy(srcr_hbm.at[wid], sidx_all)
        pltpu.sync_copy(dstr_hbm.at[wid], didx_all)
        plsc.subcore_barrier()

        for b in range(NBUF):
            pltpu.async_copy(hs_hbm.at[sidx_all.at[b]], rows[b], sems[b])

        def body(g, carry):
            j0 = g * NBUF
            for b in range(NBUF):
                j = j0 + b
                pltpu.make_async_copy(
                    hs_hbm.at[pl.ds(0, C)], rows[b], sems[b]).wait()
                pltpu.sync_copy(rows[b], acc_sh.at[didx_all.at[j]], add=True)
                pltpu.async_copy(
                    hs_hbm.at[sidx_all.at[j + NBUF]], rows[b], sems[b])
            return carry

        lax.fori_loop(0, NCHUNK // NBUF - 1, body, 0)

        jt = NCHUNK - NBUF
        for b in range(NBUF):
            pltpu.make_async_copy(hs_hbm.at[pl.ds(0, C)], rows[b], sems[b]).wait()
            pltpu.sync_copy(rows[b], acc_sh.at[didx_all.at[jt + b]], add=True)

        plsc.subcore_barrier()
        pltpu.sync_copy(acc_sh.at[pl.ds(r0, RPT)], out_hbm.at[c, pl.ds(r0, RPT)])

        @pl.when(s == 0)
        def _():
            pltpu.sync_copy(acc_sh.at[pl.ds(NS * RPT, RREM)],
                            out_hbm.at[c, pl.ds(NS * RPT, RREM)])

    return k(hs, srcr, dstr)


BLK = 1000


def _tc_first(deg_t, x, w1):
    def body(deg_ref, x_ref, w_ref, hs_ref, dinv_ref):
        deg = deg_ref[:, 0:1] + deg_ref[:, 1:2] + 1.0
        dinv = lax.rsqrt(deg)
        hw = jnp.dot(x_ref[...], w_ref[...], preferred_element_type=jnp.float32)
        hs_ref[...] = hw * dinv
        dinv_ref[...] = dinv

    return pl.pallas_call(
        body,
        grid=(N // BLK,),
        in_specs=[
            pl.BlockSpec((BLK, NC), lambda i: (i, 0)),
            pl.BlockSpec((BLK, D_IN), lambda i: (i, 0)),
            pl.BlockSpec((D_IN, HID), lambda i: (0, 0)),
        ],
        out_specs=[
            pl.BlockSpec((BLK, HID), lambda i: (i, 0)),
            pl.BlockSpec((BLK, 1), lambda i: (i, 0)),
        ],
        out_shape=[
            jax.ShapeDtypeStruct((N, HID), jnp.float32),
            jax.ShapeDtypeStruct((N, 1), jnp.float32),
        ],
    )(deg_t, x, w1)


def _tc_mid(acc, hs, dinv, b_row, w_next):
    def body(acc_ref, hs_ref, dinv_ref, b_ref, w_ref, out_ref):
        t = acc_ref[0] + acc_ref[1] - hs_ref[...]
        h = jnp.maximum(t * dinv_ref[...] + b_ref[...], 0.0)
        hw = jnp.dot(h, w_ref[...], preferred_element_type=jnp.float32)
        out_ref[...] = hw * dinv_ref[...]

    return pl.pallas_call(
        body,
        grid=(N // BLK,),
        in_specs=[
            pl.BlockSpec((NC, BLK, HID), lambda i: (0, i, 0)),
            pl.BlockSpec((BLK, HID), lambda i: (i, 0)),
            pl.BlockSpec((BLK, 1), lambda i: (i, 0)),
            pl.BlockSpec((1, HID), lambda i: (0, 0)),
            pl.BlockSpec((HID, HID), lambda i: (0, 0)),
        ],
        out_specs=pl.BlockSpec((BLK, HID), lambda i: (i, 0)),
        out_shape=jax.ShapeDtypeStruct((N, HID), jnp.float32),
    )(acc, hs, dinv, b_row, w_next)


def _tc_last(acc, hs, dinv, b_row, fc_w, fc_b_row):
    def body(acc_ref, hs_ref, dinv_ref, b_ref, fcw_ref, fcb_ref, out_ref):
        t = acc_ref[0] + acc_ref[1] - hs_ref[...]
        h = jnp.maximum(t * dinv_ref[...] + b_ref[...], 0.0)
        out_ref[...] = (
            jnp.dot(h, fcw_ref[...], preferred_element_type=jnp.float32)
            + fcb_ref[...]
        )

    return pl.pallas_call(
        body,
        grid=(N // BLK,),
        in_specs=[
            pl.BlockSpec((NC, BLK, HID), lambda i: (0, i, 0)),
            pl.BlockSpec((BLK, HID), lambda i: (i, 0)),
            pl.BlockSpec((BLK, 1), lambda i: (i, 0)),
            pl.BlockSpec((1, HID), lambda i: (0, 0)),
            pl.BlockSpec((HID, 1), lambda i: (0, 0)),
            pl.BlockSpec((1, 1), lambda i: (0, 0)),
        ],
        out_specs=pl.BlockSpec((BLK, 1), lambda i: (i, 0)),
        out_shape=jax.ShapeDtypeStruct((N, 1), jnp.float32),
    )(acc, hs, dinv, b_row, fc_w, fc_b_row)


def kernel(x, edge_index, W1, b1, W2, b2, W3, b3, fc_w, fc_b):
    srcr = edge_index[0].reshape(NW, NCHUNK, C)
    dstr = edge_index[1].reshape(NW, NCHUNK, C)
    zeros_n = jnp.zeros((N,), jnp.float32)

    deg2 = _sc_degree(dstr, zeros_n)
    deg_t = deg2.T
    hs1, dinv = _tc_first(deg_t, x, W1)

    acc1 = _sc_spmm(hs1, srcr, dstr)
    hs2 = _tc_mid(acc1, hs1, dinv, b1.reshape(1, HID), W2)
    acc2 = _sc_spmm(hs2, srcr, dstr)
    hs3 = _tc_mid(acc2, hs2, dinv, b2.reshape(1, HID), W3)
    acc3 = _sc_spmm(hs3, srcr, dstr)
    return _tc_last(acc3, hs3, dinv, b3.reshape(1, HID),
                    fc_w, fc_b.reshape(1, 1))
